# Initial kernel scaffold; baseline (speedup 1.0000x reference)
import jax
import jax.numpy as jnp
from jax.experimental import pallas as pl


def kernel(x1, x2, x3, item_emb, user_emb, W1, b1, W2, b2, W3, b3, W4, b4, Wo, bo, item_emb1, user_emb1, Wf, bf):
    raise NotImplementedError("write your pallas kernel here")


# speedup vs baseline: 1.3070x; 1.3070x over previous
import functools

import jax
import jax.numpy as jnp
from jax import lax
from jax.experimental import pallas as pl
from jax.experimental.pallas import tpu as pltpu
from jax.experimental.pallas import tpu_sc as plsc

B = 16384
D_IV = 128
D_GMF = 64
D_SIDE = 358
BM = 512
NC = 2
NS = 16
NW = NC * NS
BPW = B // NW
CH = 128
NCH = BPW // CH

@functools.lru_cache(maxsize=1)
def _build_sc_gather():
  mesh = plsc.VectorSubcoreMesh(core_axis_name="c", subcore_axis_name="s")

  @functools.partial(
      pl.kernel,
      out_type=[
          jax.ShapeDtypeStruct((B, 2 * D_IV), jnp.float32),
          jax.ShapeDtypeStruct((B, 2 * D_GMF), jnp.float32),
          jax.ShapeDtypeStruct((B, 2 * D_GMF), jnp.float32),
      ],
      mesh=mesh,
      scratch_types=[
        pltpu.VMEM((CH,), jnp.int32),
        pltpu.VMEM((CH,), jnp.int32),
        pltpu.VMEM((CH, D_IV), jnp.float32),
        pltpu.VMEM((CH, D_IV), jnp.float32),
        pltpu.VMEM((CH, 2 * D_GMF), jnp.float32),
        pltpu.VMEM((CH, 2 * D_GMF), jnp.float32),
          pltpu.SemaphoreType.DMA,
          pltpu.SemaphoreType.DMA,
          pltpu.SemaphoreType.DMA,
          pltpu.SemaphoreType.DMA,
      ],
  )
  def sc_gather(x1_hbm, x2_hbm, item_hbm, user_hbm, gtab_hbm,
                xiu_out, g1_out, g2_out,
                idx1_v, idx2_v, iv_v, uv_v, g1_v, g2_v, s1, s2, s3, s4):
    wid = lax.axis_index("s") * NC + lax.axis_index("c")
    base = wid * BPW
    for c in range(NCH):
      off = base + c * CH
      pltpu.sync_copy(x1_hbm.at[pl.ds(off, CH)], idx1_v)
      pltpu.sync_copy(x2_hbm.at[pl.ds(off, CH)], idx2_v)
      cp1 = pltpu.async_copy(item_hbm.at[idx1_v], iv_v, s1)
      cp2 = pltpu.async_copy(user_hbm.at[idx2_v], uv_v, s2)
      cp3 = pltpu.async_copy(gtab_hbm.at[idx1_v], g1_v, s3)
      cp4 = pltpu.async_copy(gtab_hbm.at[idx2_v], g2_v, s4)
      cp1.wait()
      pltpu.sync_copy(iv_v, xiu_out.at[pl.ds(off, CH), pl.ds(0, D_IV)])
      cp2.wait()
      pltpu.sync_copy(uv_v, xiu_out.at[pl.ds(off, CH), pl.ds(D_IV, D_IV)])
      cp3.wait()
      pltpu.sync_copy(g1_v, g1_out.at[pl.ds(off, CH)])
      cp4.wait()
      pltpu.sync_copy(g2_v, g2_out.at[pl.ds(off, CH)])

  return sc_gather


def _mlp_body(xiu_ref, x3_ref, giv_ref, guv_ref,
              w1ab_ref, w1c_ref, b1_ref, w2_ref, b2_ref, w3_ref, b3_ref,
              w4_ref, b4_ref, wo_ref, wfg_ref, c0_ref, out_ref):
  f32 = jnp.float32
  bf16 = jnp.bfloat16
  h = jnp.dot(xiu_ref[...].astype(bf16), w1ab_ref[...],
              preferred_element_type=f32)
  h = h + jnp.dot(x3_ref[...].astype(bf16), w1c_ref[...],
                  preferred_element_type=f32)
  h = jnp.maximum(h + b1_ref[...], 0.0).astype(bf16)
  h = jnp.maximum(jnp.dot(h, w2_ref[...], preferred_element_type=f32)
                  + b2_ref[...], 0.0).astype(bf16)
  h = jnp.maximum(jnp.dot(h, w3_ref[...], preferred_element_type=f32)
                  + b3_ref[...], 0.0).astype(bf16)
  h = jnp.maximum(jnp.dot(h, w4_ref[...], preferred_element_type=f32)
                  + b4_ref[...], 0.0)
  mlp = jnp.sum(h * wo_ref[...], axis=1, keepdims=True)
  giv = giv_ref[...][:, :D_GMF]
  guv = guv_ref[...][:, D_GMF:]
  g = jnp.sum(giv * guv * wfg_ref[...], axis=1, keepdims=True)
  z = mlp + g + c0_ref[...]
  out_ref[...] = 1.0 / (1.0 + jnp.exp(-z))


def _tc_mlp(xiu, x3, giv, guv, w1ab, w1c, b1r, w2, b2r, w3, b3r, w4, b4r,
            wo_s, wfg, c0, interpret=False):
  h1, h2, h3, h4 = 1024, 1024, 512, 256
  return pl.pallas_call(
      _mlp_body,
      grid=(B // BM,),
      in_specs=[
          pl.BlockSpec((BM, 2 * D_IV), lambda i: (i, 0)),
          pl.BlockSpec((BM, D_SIDE), lambda i: (i, 0)),
          pl.BlockSpec((BM, 2 * D_GMF), lambda i: (i, 0)),
          pl.BlockSpec((BM, 2 * D_GMF), lambda i: (i, 0)),
          pl.BlockSpec((2 * D_IV, h1), lambda i: (0, 0)),
          pl.BlockSpec((D_SIDE, h1), lambda i: (0, 0)),
          pl.BlockSpec((1, h1), lambda i: (0, 0)),
          pl.BlockSpec((h1, h2), lambda i: (0, 0)),
          pl.BlockSpec((1, h2), lambda i: (0, 0)),
          pl.BlockSpec((h2, h3), lambda i: (0, 0)),
          pl.BlockSpec((1, h3), lambda i: (0, 0)),
          pl.BlockSpec((h3, h4), lambda i: (0, 0)),
          pl.BlockSpec((1, h4), lambda i: (0, 0)),
          pl.BlockSpec((1, h4), lambda i: (0, 0)),
          pl.BlockSpec((1, D_GMF), lambda i: (0, 0)),
          pl.BlockSpec((1, 1), lambda i: (0, 0)),
      ],
      out_specs=pl.BlockSpec((BM, 1), lambda i: (i, 0)),
      out_shape=jax.ShapeDtypeStruct((B, 1), jnp.float32),
      compiler_params=pltpu.CompilerParams(
          dimension_semantics=("arbitrary",)),
      interpret=interpret,
  )(xiu, x3, giv, guv, w1ab, w1c, b1r, w2, b2r, w3, b3r, w4, b4r,
    wo_s, wfg, c0)


def kernel(x1, x2, x3, item_emb, user_emb, W1, b1, W2, b2, W3, b3, W4, b4,
           Wo, bo, item_emb1, user_emb1, Wf, bf):
  x1 = x1.astype(jnp.int32)
  x2 = x2.astype(jnp.int32)
  gtab = jnp.concatenate([item_emb1, user_emb1], axis=1)
  xiu, giv, guv = _build_sc_gather()(x1, x2, item_emb, user_emb, gtab)
  bf16 = jnp.bfloat16
  w1ab = W1[:2 * D_IV].astype(bf16)
  w1c = W1[2 * D_IV:].astype(bf16)
  wf0 = Wf[0, 0]
  wo_s = (Wo[:, 0] * wf0)[None, :]
  wfg = Wf[1:, 0][None, :]
  c0 = (bo[0] * wf0 + bf[0])[None, None]
  return _tc_mlp(xiu, x3, giv, guv, w1ab, w1c, b1[None, :],
                 W2.astype(bf16), b2[None, :], W3.astype(bf16), b3[None, :],
                 W4.astype(bf16), b4[None, :], wo_s, wfg, c0)


# speedup vs baseline: 1.3072x; 1.0001x over previous
import functools

import jax
import jax.numpy as jnp
from jax import lax
from jax.experimental import pallas as pl
from jax.experimental.pallas import tpu as pltpu
from jax.experimental.pallas import tpu_sc as plsc

B = 16384
D_IV = 128
D_GMF = 64
D_SIDE = 358
BM = 512
NC = 2
NS = 16
NW = NC * NS
BPW = B // NW
CH = 64
NCH = BPW // CH
NB = 2

@functools.lru_cache(maxsize=1)
def _build_sc_gather():
  mesh = plsc.VectorSubcoreMesh(core_axis_name="c", subcore_axis_name="s")

  @functools.partial(
      pl.kernel,
      out_type=[
          jax.ShapeDtypeStruct((B, 2 * D_IV), jnp.float32),
          jax.ShapeDtypeStruct((B, 2 * D_GMF), jnp.float32),
          jax.ShapeDtypeStruct((B, 2 * D_GMF), jnp.float32),
      ],
      mesh=mesh,
      scratch_types=[
          pltpu.VMEM((BPW,), jnp.int32),
          pltpu.VMEM((BPW,), jnp.int32),
          pltpu.VMEM((NB, CH, D_IV), jnp.float32),
          pltpu.VMEM((NB, CH, D_IV), jnp.float32),
          pltpu.VMEM((NB, CH, 2 * D_GMF), jnp.float32),
          pltpu.VMEM((NB, CH, 2 * D_GMF), jnp.float32),
          pltpu.SemaphoreType.DMA,
          pltpu.SemaphoreType.DMA,
          pltpu.SemaphoreType.DMA,
          pltpu.SemaphoreType.DMA,
      ],
  )
  def sc_gather(x1_hbm, x2_hbm, item_hbm, user_hbm, gtab_hbm,
                xiu_out, g1_out, g2_out,
                idx1_v, idx2_v, iv_v, uv_v, g1_v, g2_v, sg0, sg1, sw0, sw1):
    wid = lax.axis_index("s") * NC + lax.axis_index("c")
    base = wid * BPW
    pltpu.sync_copy(x1_hbm.at[pl.ds(base, BPW)], idx1_v)
    pltpu.sync_copy(x2_hbm.at[pl.ds(base, BPW)], idx2_v)
    sg = (sg0, sg1)
    sw = (sw0, sw1)
    pend_g = [None] * NB
    pend_w = [None] * NB

    def fire_writes(c):
      b = c % NB
      for g in pend_g[b]:
        g.wait()
      off = base + c * CH
      w1 = pltpu.async_copy(iv_v.at[b],
                            xiu_out.at[pl.ds(off, CH), pl.ds(0, D_IV)], sw[b])
      w2 = pltpu.async_copy(uv_v.at[b],
                            xiu_out.at[pl.ds(off, CH), pl.ds(D_IV, D_IV)],
                            sw[b])
      w3 = pltpu.async_copy(g1_v.at[b], g1_out.at[pl.ds(off, CH)], sw[b])
      w4 = pltpu.async_copy(g2_v.at[b], g2_out.at[pl.ds(off, CH)], sw[b])
      pend_w[b] = (w1, w2, w3, w4)

    for c in range(NCH):
      b = c % NB
      if pend_w[b] is not None:
        for w in pend_w[b]:
          w.wait()
      loc = c * CH
      pend_g[b] = (
          pltpu.async_copy(item_hbm.at[idx1_v.at[pl.ds(loc, CH)]],
                           iv_v.at[b], sg[b]),
          pltpu.async_copy(user_hbm.at[idx2_v.at[pl.ds(loc, CH)]],
                           uv_v.at[b], sg[b]),
          pltpu.async_copy(gtab_hbm.at[idx1_v.at[pl.ds(loc, CH)]],
                           g1_v.at[b], sg[b]),
          pltpu.async_copy(gtab_hbm.at[idx2_v.at[pl.ds(loc, CH)]],
                           g2_v.at[b], sg[b]),
      )
      if c >= 1:
        fire_writes(c - 1)
    fire_writes(NCH - 1)
    for pw in pend_w:
      if pw is not None:
        for w in pw:
          w.wait()

  return sc_gather


def _mlp_body(xiu_ref, x3_ref, giv_ref, guv_ref,
              w1ab_ref, w1c_ref, b1_ref, w2_ref, b2_ref, w3_ref, b3_ref,
              w4_ref, b4_ref, wo_ref, wfg_ref, c0_ref, out_ref):
  f32 = jnp.float32
  bf16 = jnp.bfloat16
  h = jnp.dot(xiu_ref[...].astype(bf16), w1ab_ref[...],
              preferred_element_type=f32)
  h = h + jnp.dot(x3_ref[...].astype(bf16), w1c_ref[...],
                  preferred_element_type=f32)
  h = jnp.maximum(h + b1_ref[...], 0.0).astype(bf16)
  h = jnp.maximum(jnp.dot(h, w2_ref[...], preferred_element_type=f32)
                  + b2_ref[...], 0.0).astype(bf16)
  h = jnp.maximum(jnp.dot(h, w3_ref[...], preferred_element_type=f32)
                  + b3_ref[...], 0.0).astype(bf16)
  h = jnp.maximum(jnp.dot(h, w4_ref[...], preferred_element_type=f32)
                  + b4_ref[...], 0.0)
  mlp = jnp.sum(h * wo_ref[...], axis=1, keepdims=True)
  giv = giv_ref[...][:, :D_GMF]
  guv = guv_ref[...][:, D_GMF:]
  g = jnp.sum(giv * guv * wfg_ref[...], axis=1, keepdims=True)
  z = mlp + g + c0_ref[...]
  out_ref[...] = 1.0 / (1.0 + jnp.exp(-z))


def _tc_mlp(xiu, x3, giv, guv, w1ab, w1c, b1r, w2, b2r, w3, b3r, w4, b4r,
            wo_s, wfg, c0, interpret=False):
  h1, h2, h3, h4 = 1024, 1024, 512, 256
  return pl.pallas_call(
      _mlp_body,
      grid=(B // BM,),
      in_specs=[
          pl.BlockSpec((BM, 2 * D_IV), lambda i: (i, 0)),
          pl.BlockSpec((BM, D_SIDE), lambda i: (i, 0)),
          pl.BlockSpec((BM, 2 * D_GMF), lambda i: (i, 0)),
          pl.BlockSpec((BM, 2 * D_GMF), lambda i: (i, 0)),
          pl.BlockSpec((2 * D_IV, h1), lambda i: (0, 0)),
          pl.BlockSpec((D_SIDE, h1), lambda i: (0, 0)),
          pl.BlockSpec((1, h1), lambda i: (0, 0)),
          pl.BlockSpec((h1, h2), lambda i: (0, 0)),
          pl.BlockSpec((1, h2), lambda i: (0, 0)),
          pl.BlockSpec((h2, h3), lambda i: (0, 0)),
          pl.BlockSpec((1, h3), lambda i: (0, 0)),
          pl.BlockSpec((h3, h4), lambda i: (0, 0)),
          pl.BlockSpec((1, h4), lambda i: (0, 0)),
          pl.BlockSpec((1, h4), lambda i: (0, 0)),
          pl.BlockSpec((1, D_GMF), lambda i: (0, 0)),
          pl.BlockSpec((1, 1), lambda i: (0, 0)),
      ],
      out_specs=pl.BlockSpec((BM, 1), lambda i: (i, 0)),
      out_shape=jax.ShapeDtypeStruct((B, 1), jnp.float32),
      compiler_params=pltpu.CompilerParams(
          dimension_semantics=("arbitrary",)),
      interpret=interpret,
  )(xiu, x3, giv, guv, w1ab, w1c, b1r, w2, b2r, w3, b3r, w4, b4r,
    wo_s, wfg, c0)


def kernel(x1, x2, x3, item_emb, user_emb, W1, b1, W2, b2, W3, b3, W4, b4,
           Wo, bo, item_emb1, user_emb1, Wf, bf):
  x1 = x1.astype(jnp.int32)
  x2 = x2.astype(jnp.int32)
  gtab = jnp.concatenate([item_emb1, user_emb1], axis=1)
  xiu, giv, guv = _build_sc_gather()(x1, x2, item_emb, user_emb, gtab)
  bf16 = jnp.bfloat16
  w1ab = W1[:2 * D_IV].astype(bf16)
  w1c = W1[2 * D_IV:].astype(bf16)
  wf0 = Wf[0, 0]
  wo_s = (Wo[:, 0] * wf0)[None, :]
  wfg = Wf[1:, 0][None, :]
  c0 = (bo[0] * wf0 + bf[0])[None, None]
  return _tc_mlp(xiu, x3, giv, guv, w1ab, w1c, b1[None, :],
                 W2.astype(bf16), b2[None, :], W3.astype(bf16), b3[None, :],
                 W4.astype(bf16), b4[None, :], wo_s, wfg, c0)


# speedup vs baseline: 1.3208x; 1.0105x over previous
import functools

import jax
import jax.numpy as jnp
from jax import lax
from jax.experimental import pallas as pl
from jax.experimental.pallas import tpu as pltpu
from jax.experimental.pallas import tpu_sc as plsc

B = 16384
N_TAB = 100000
D_IV = 128
D_GMF = 64
D_SIDE = 358
BM = 512
NC = 2
NS = 16
NW = NC * NS
BPW = B // NW
CH = 64
NCH = BPW // CH
NB = 2
RT = 500
RJ = 8


def _transcat_body(a_ref, b_ref, o_ref):
  for j in range(RJ):
    a = jnp.transpose(a_ref[:, j, :])
    b = jnp.transpose(b_ref[:, j, :])
    o_ref[0, pl.ds(j * RT, RT), :] = jnp.concatenate([a, b], axis=1)


def _transcat(it1_t, us1_t):
  a3 = it1_t.reshape(D_GMF, N_TAB // RT, RT)
  b3 = us1_t.reshape(D_GMF, N_TAB // RT, RT)
  nsteps = N_TAB // (RT * RJ)
  out = pl.pallas_call(
      _transcat_body,
      grid=(nsteps,),
      in_specs=[pl.BlockSpec((D_GMF, RJ, RT), lambda i: (0, i, 0)),
                pl.BlockSpec((D_GMF, RJ, RT), lambda i: (0, i, 0))],
      out_specs=pl.BlockSpec((1, RT * RJ, 2 * D_GMF), lambda i: (i, 0, 0)),
      out_shape=jax.ShapeDtypeStruct((nsteps, RT * RJ, 2 * D_GMF),
                                     jnp.float32),
      compiler_params=pltpu.CompilerParams(
          dimension_semantics=("arbitrary",)),
  )(a3, b3)
  return out.reshape(N_TAB, 2 * D_GMF)


def _sc_pipeline(gather_fn, write_fn):
  pend_g = [None] * NB
  pend_w = [None] * NB

  def fire_writes(c):
    b = c % NB
    for g in pend_g[b]:
      g.wait()
    pend_w[b] = write_fn(c, b)

  for c in range(NCH):
    b = c % NB
    if pend_w[b] is not None:
      for w in pend_w[b]:
        w.wait()
    pend_g[b] = gather_fn(c, b)
    if c >= 1:
      fire_writes(c - 1)
  fire_writes(NCH - 1)
  for pw in pend_w:
    if pw is not None:
      for w in pw:
        w.wait()


@functools.lru_cache(maxsize=1)
def _build_sc_k1():
  mesh = plsc.VectorSubcoreMesh(core_axis_name="c", subcore_axis_name="s")

  @functools.partial(
      pl.kernel,
      out_type=jax.ShapeDtypeStruct((B, 2 * D_IV), jnp.float32),
      mesh=mesh,
      scratch_types=[
          pltpu.VMEM((BPW,), jnp.int32),
          pltpu.VMEM((BPW,), jnp.int32),
          pltpu.VMEM((NB, CH, D_IV), jnp.float32),
          pltpu.VMEM((NB, CH, D_IV), jnp.float32),
          pltpu.SemaphoreType.DMA,
          pltpu.SemaphoreType.DMA,
      ],
  )
  def k1(x1_hbm, x2_hbm, item_hbm, user_hbm, xiu_out,
         idx1_v, idx2_v, iv_v, uv_v, sg, sw):
    wid = lax.axis_index("s") * NC + lax.axis_index("c")
    base = wid * BPW
    pltpu.sync_copy(x1_hbm.at[pl.ds(base, BPW)], idx1_v)
    pltpu.sync_copy(x2_hbm.at[pl.ds(base, BPW)], idx2_v)

    def gather(c, b):
      loc = c * CH
      return (
          pltpu.async_copy(item_hbm.at[idx1_v.at[pl.ds(loc, CH)]],
                           iv_v.at[b], sg),
          pltpu.async_copy(user_hbm.at[idx2_v.at[pl.ds(loc, CH)]],
                           uv_v.at[b], sg),
      )

    def write(c, b):
      off = base + c * CH
      return (
          pltpu.async_copy(iv_v.at[b],
                           xiu_out.at[pl.ds(off, CH), pl.ds(0, D_IV)], sw),
          pltpu.async_copy(uv_v.at[b],
                           xiu_out.at[pl.ds(off, CH), pl.ds(D_IV, D_IV)],
                           sw),
      )

    _sc_pipeline(gather, write)

  return k1


@functools.lru_cache(maxsize=1)
def _build_sc_k2():
  mesh = plsc.VectorSubcoreMesh(core_axis_name="c", subcore_axis_name="s")

  @functools.partial(
      pl.kernel,
      out_type=[
          jax.ShapeDtypeStruct((B, 2 * D_GMF), jnp.float32),
          jax.ShapeDtypeStruct((B, 2 * D_GMF), jnp.float32),
      ],
      mesh=mesh,
      scratch_types=[
          pltpu.VMEM((BPW,), jnp.int32),
          pltpu.VMEM((BPW,), jnp.int32),
          pltpu.VMEM((NB, CH, 2 * D_GMF), jnp.float32),
          pltpu.VMEM((NB, CH, 2 * D_GMF), jnp.float32),
          pltpu.SemaphoreType.DMA,
          pltpu.SemaphoreType.DMA,
      ],
  )
  def k2(x1_hbm, x2_hbm, gtab_hbm, g1_out, g2_out,
         idx1_v, idx2_v, g1_v, g2_v, sg, sw):
    wid = lax.axis_index("s") * NC + lax.axis_index("c")
    base = wid * BPW
    pltpu.sync_copy(x1_hbm.at[pl.ds(base, BPW)], idx1_v)
    pltpu.sync_copy(x2_hbm.at[pl.ds(base, BPW)], idx2_v)

    def gather(c, b):
      loc = c * CH
      return (
          pltpu.async_copy(gtab_hbm.at[idx1_v.at[pl.ds(loc, CH)]],
                           g1_v.at[b], sg),
          pltpu.async_copy(gtab_hbm.at[idx2_v.at[pl.ds(loc, CH)]],
                           g2_v.at[b], sg),
      )

    def write(c, b):
      off = base + c * CH
      return (
          pltpu.async_copy(g1_v.at[b], g1_out.at[pl.ds(off, CH)], sw),
          pltpu.async_copy(g2_v.at[b], g2_out.at[pl.ds(off, CH)], sw),
      )

    _sc_pipeline(gather, write)

  return k2


def _mlp_body(xiu_ref, x3t_ref, g1_ref, g2_ref,
              w1ab_ref, w1c_ref, b1_ref, w2_ref, b2_ref, w3_ref, b3_ref,
              w4_ref, b4_ref, wo_ref, wfg_ref, c0_ref, out_ref):
  f32 = jnp.float32
  bf16 = jnp.bfloat16
  h = jnp.dot(xiu_ref[...].astype(bf16), w1ab_ref[...],
              preferred_element_type=f32)
  h = h + lax.dot_general(x3t_ref[...].astype(bf16), w1c_ref[...],
                          (((0,), (0,)), ((), ())),
                          preferred_element_type=f32)
  h = jnp.maximum(h + b1_ref[...], 0.0).astype(bf16)
  h = jnp.maximum(jnp.dot(h, w2_ref[...], preferred_element_type=f32)
                  + b2_ref[...], 0.0).astype(bf16)
  h = jnp.maximum(jnp.dot(h, w3_ref[...], preferred_element_type=f32)
                  + b3_ref[...], 0.0).astype(bf16)
  h = jnp.maximum(jnp.dot(h, w4_ref[...], preferred_element_type=f32)
                  + b4_ref[...], 0.0)
  mlp = jnp.sum(h * wo_ref[...], axis=1, keepdims=True)
  giv = g1_ref[...][:, :D_GMF]
  guv = g2_ref[...][:, D_GMF:]
  g = jnp.sum(giv * guv * wfg_ref[...], axis=1, keepdims=True)
  z = mlp + g + c0_ref[...]
  out_ref[...] = jnp.transpose(1.0 / (1.0 + jnp.exp(-z)))


def _tc_mlp(xiu, x3t, g1, g2, w1ab, w1c, b1r, w2, b2r, w3, b3r, w4, b4r,
            wo_s, wfg, c0, interpret=False):
  h1, h2, h3, h4 = 1024, 1024, 512, 256
  return pl.pallas_call(
      _mlp_body,
      grid=(B // BM,),
      in_specs=[
          pl.BlockSpec((BM, 2 * D_IV), lambda i: (i, 0)),
          pl.BlockSpec((D_SIDE, BM), lambda i: (0, i)),
          pl.BlockSpec((BM, 2 * D_GMF), lambda i: (i, 0)),
          pl.BlockSpec((BM, 2 * D_GMF), lambda i: (i, 0)),
          pl.BlockSpec((2 * D_IV, h1), lambda i: (0, 0)),
          pl.BlockSpec((D_SIDE, h1), lambda i: (0, 0)),
          pl.BlockSpec((1, h1), lambda i: (0, 0)),
          pl.BlockSpec((h1, h2), lambda i: (0, 0)),
          pl.BlockSpec((1, h2), lambda i: (0, 0)),
          pl.BlockSpec((h2, h3), lambda i: (0, 0)),
          pl.BlockSpec((1, h3), lambda i: (0, 0)),
          pl.BlockSpec((h3, h4), lambda i: (0, 0)),
          pl.BlockSpec((1, h4), lambda i: (0, 0)),
          pl.BlockSpec((1, h4), lambda i: (0, 0)),
          pl.BlockSpec((1, D_GMF), lambda i: (0, 0)),
          pl.BlockSpec((1, 1), lambda i: (0, 0)),
      ],
      out_specs=pl.BlockSpec((1, BM), lambda i: (0, i)),
      out_shape=jax.ShapeDtypeStruct((1, B), jnp.float32),
      compiler_params=pltpu.CompilerParams(
          dimension_semantics=("arbitrary",)),
      interpret=interpret,
  )(xiu, x3t, g1, g2, w1ab, w1c, b1r, w2, b2r, w3, b3r, w4, b4r,
    wo_s, wfg, c0)


def kernel(x1, x2, x3, item_emb, user_emb, W1, b1, W2, b2, W3, b3, W4, b4,
           Wo, bo, item_emb1, user_emb1, Wf, bf):
  x1 = x1.astype(jnp.int32)
  x2 = x2.astype(jnp.int32)
  gtab = _transcat(item_emb1.T, user_emb1.T)
  xiu = _build_sc_k1()(x1, x2, item_emb, user_emb)
  g1, g2 = _build_sc_k2()(x1, x2, gtab)
  bf16 = jnp.bfloat16
  w1ab = W1[:2 * D_IV].astype(bf16)
  w1c = W1[2 * D_IV:].astype(bf16)
  wf0 = Wf[0, 0]
  wo_s = (Wo[:, 0] * wf0)[None, :]
  wfg = Wf[1:, 0][None, :]
  c0 = (bo[0] * wf0 + bf[0])[None, None]
  out = _tc_mlp(xiu, x3.T, g1, g2, w1ab, w1c, b1[None, :],
                W2.astype(bf16), b2[None, :], W3.astype(bf16), b3[None, :],
                W4.astype(bf16), b4[None, :], wo_s, wfg, c0)
  return out.T


# speedup vs baseline: 1.5940x; 1.2068x over previous
import functools

import jax
import jax.numpy as jnp
from jax import lax
from jax.experimental import pallas as pl
from jax.experimental.pallas import tpu as pltpu
from jax.experimental.pallas import tpu_sc as plsc

B = 16384
N_TAB = 100000
D_IV = 128
D_GMF = 64
D_SIDE = 358
BM = 512
NC = 2
NS = 16
NW = NC * NS
BPW = B // NW
CH = 64
NCH = BPW // CH
NB = 2
RT = 2048


def _transcat_body(a_ref, b_ref, o_ref):
  a = jnp.transpose(a_ref[...])
  b = jnp.transpose(b_ref[...])
  o_ref[...] = jnp.concatenate([a, b], axis=1)


def _transcat(it1_t, us1_t):
  nsteps = (N_TAB + RT - 1) // RT
  return pl.pallas_call(
      _transcat_body,
      grid=(nsteps,),
      in_specs=[pl.BlockSpec((D_GMF, RT), lambda i: (0, i)),
                pl.BlockSpec((D_GMF, RT), lambda i: (0, i))],
      out_specs=pl.BlockSpec((RT, 2 * D_GMF), lambda i: (i, 0)),
      out_shape=jax.ShapeDtypeStruct((N_TAB, 2 * D_GMF), jnp.float32),
      compiler_params=pltpu.CompilerParams(
          dimension_semantics=("arbitrary",)),
  )(it1_t, us1_t)


def _sc_pipeline(gather_fn, write_fn):
  pend_g = [None] * NB
  pend_w = [None] * NB

  def fire_writes(c):
    b = c % NB
    for g in pend_g[b]:
      g.wait()
    pend_w[b] = write_fn(c, b)

  for c in range(NCH):
    b = c % NB
    if pend_w[b] is not None:
      for w in pend_w[b]:
        w.wait()
    pend_g[b] = gather_fn(c, b)
    if c >= 1:
      fire_writes(c - 1)
  fire_writes(NCH - 1)
  for pw in pend_w:
    if pw is not None:
      for w in pw:
        w.wait()


@functools.lru_cache(maxsize=1)
def _build_sc_k1():
  mesh = plsc.VectorSubcoreMesh(core_axis_name="c", subcore_axis_name="s")

  @functools.partial(
      pl.kernel,
      out_type=jax.ShapeDtypeStruct((B, 2 * D_IV), jnp.float32),
      mesh=mesh,
      scratch_types=[
          pltpu.VMEM((BPW,), jnp.int32),
          pltpu.VMEM((BPW,), jnp.int32),
          pltpu.VMEM((NB, CH, D_IV), jnp.float32),
          pltpu.VMEM((NB, CH, D_IV), jnp.float32),
          pltpu.SemaphoreType.DMA,
          pltpu.SemaphoreType.DMA,
      ],
  )
  def k1(x1_hbm, x2_hbm, item_hbm, user_hbm, xiu_out,
         idx1_v, idx2_v, iv_v, uv_v, sg, sw):
    wid = lax.axis_index("s") * NC + lax.axis_index("c")
    base = wid * BPW
    pltpu.sync_copy(x1_hbm.at[pl.ds(base, BPW)], idx1_v)
    pltpu.sync_copy(x2_hbm.at[pl.ds(base, BPW)], idx2_v)

    def gather(c, b):
      loc = c * CH
      return (
          pltpu.async_copy(item_hbm.at[idx1_v.at[pl.ds(loc, CH)]],
                           iv_v.at[b], sg),
          pltpu.async_copy(user_hbm.at[idx2_v.at[pl.ds(loc, CH)]],
                           uv_v.at[b], sg),
      )

    def write(c, b):
      off = base + c * CH
      return (
          pltpu.async_copy(iv_v.at[b],
                           xiu_out.at[pl.ds(off, CH), pl.ds(0, D_IV)], sw),
          pltpu.async_copy(uv_v.at[b],
                           xiu_out.at[pl.ds(off, CH), pl.ds(D_IV, D_IV)],
                           sw),
      )

    _sc_pipeline(gather, write)

  return k1


@functools.lru_cache(maxsize=1)
def _build_sc_k2():
  mesh = plsc.VectorSubcoreMesh(core_axis_name="c", subcore_axis_name="s")

  @functools.partial(
      pl.kernel,
      out_type=[
          jax.ShapeDtypeStruct((B, 2 * D_GMF), jnp.float32),
          jax.ShapeDtypeStruct((B, 2 * D_GMF), jnp.float32),
      ],
      mesh=mesh,
      scratch_types=[
          pltpu.VMEM((BPW,), jnp.int32),
          pltpu.VMEM((BPW,), jnp.int32),
          pltpu.VMEM((NB, CH, 2 * D_GMF), jnp.float32),
          pltpu.VMEM((NB, CH, 2 * D_GMF), jnp.float32),
          pltpu.SemaphoreType.DMA,
          pltpu.SemaphoreType.DMA,
      ],
  )
  def k2(x1_hbm, x2_hbm, gtab_hbm, g1_out, g2_out,
         idx1_v, idx2_v, g1_v, g2_v, sg, sw):
    wid = lax.axis_index("s") * NC + lax.axis_index("c")
    base = wid * BPW
    pltpu.sync_copy(x1_hbm.at[pl.ds(base, BPW)], idx1_v)
    pltpu.sync_copy(x2_hbm.at[pl.ds(base, BPW)], idx2_v)

    def gather(c, b):
      loc = c * CH
      return (
          pltpu.async_copy(gtab_hbm.at[idx1_v.at[pl.ds(loc, CH)]],
                           g1_v.at[b], sg),
          pltpu.async_copy(gtab_hbm.at[idx2_v.at[pl.ds(loc, CH)]],
                           g2_v.at[b], sg),
      )

    def write(c, b):
      off = base + c * CH
      return (
          pltpu.async_copy(g1_v.at[b], g1_out.at[pl.ds(off, CH)], sw),
          pltpu.async_copy(g2_v.at[b], g2_out.at[pl.ds(off, CH)], sw),
      )

    _sc_pipeline(gather, write)

  return k2


def _mlp_body(xiu_ref, x3t_ref, g1_ref, g2_ref,
              w1ab_ref, w1c_ref, b1_ref, w2_ref, b2_ref, w3_ref, b3_ref,
              w4_ref, b4_ref, wo_ref, wfg_ref, c0_ref, out_ref):
  f32 = jnp.float32
  bf16 = jnp.bfloat16
  h = jnp.dot(xiu_ref[...].astype(bf16), w1ab_ref[...],
              preferred_element_type=f32)
  h = h + lax.dot_general(x3t_ref[...].astype(bf16), w1c_ref[...],
                          (((0,), (0,)), ((), ())),
                          preferred_element_type=f32)
  h = jnp.maximum(h + b1_ref[...], 0.0).astype(bf16)
  h = jnp.maximum(jnp.dot(h, w2_ref[...], preferred_element_type=f32)
                  + b2_ref[...], 0.0).astype(bf16)
  h = jnp.maximum(jnp.dot(h, w3_ref[...], preferred_element_type=f32)
                  + b3_ref[...], 0.0).astype(bf16)
  h = jnp.maximum(jnp.dot(h, w4_ref[...], preferred_element_type=f32)
                  + b4_ref[...], 0.0)
  mlp = jnp.sum(h * wo_ref[...], axis=1, keepdims=True)
  giv = g1_ref[...][:, :D_GMF]
  guv = g2_ref[...][:, D_GMF:]
  g = jnp.sum(giv * guv * wfg_ref[...], axis=1, keepdims=True)
  z = mlp + g + c0_ref[...]
  out_ref[...] = jnp.transpose(1.0 / (1.0 + jnp.exp(-z)))


def _tc_mlp(xiu, x3t, g1, g2, w1ab, w1c, b1r, w2, b2r, w3, b3r, w4, b4r,
            wo_s, wfg, c0, interpret=False):
  h1, h2, h3, h4 = 1024, 1024, 512, 256
  return pl.pallas_call(
      _mlp_body,
      grid=(B // BM,),
      in_specs=[
          pl.BlockSpec((BM, 2 * D_IV), lambda i: (i, 0)),
          pl.BlockSpec((D_SIDE, BM), lambda i: (0, i)),
          pl.BlockSpec((BM, 2 * D_GMF), lambda i: (i, 0)),
          pl.BlockSpec((BM, 2 * D_GMF), lambda i: (i, 0)),
          pl.BlockSpec((2 * D_IV, h1), lambda i: (0, 0)),
          pl.BlockSpec((D_SIDE, h1), lambda i: (0, 0)),
          pl.BlockSpec((1, h1), lambda i: (0, 0)),
          pl.BlockSpec((h1, h2), lambda i: (0, 0)),
          pl.BlockSpec((1, h2), lambda i: (0, 0)),
          pl.BlockSpec((h2, h3), lambda i: (0, 0)),
          pl.BlockSpec((1, h3), lambda i: (0, 0)),
          pl.BlockSpec((h3, h4), lambda i: (0, 0)),
          pl.BlockSpec((1, h4), lambda i: (0, 0)),
          pl.BlockSpec((1, h4), lambda i: (0, 0)),
          pl.BlockSpec((1, D_GMF), lambda i: (0, 0)),
          pl.BlockSpec((1, 1), lambda i: (0, 0)),
      ],
      out_specs=pl.BlockSpec((1, BM), lambda i: (0, i)),
      out_shape=jax.ShapeDtypeStruct((1, B), jnp.float32),
      compiler_params=pltpu.CompilerParams(
          dimension_semantics=("arbitrary",)),
      interpret=interpret,
  )(xiu, x3t, g1, g2, w1ab, w1c, b1r, w2, b2r, w3, b3r, w4, b4r,
    wo_s, wfg, c0)


def kernel(x1, x2, x3, item_emb, user_emb, W1, b1, W2, b2, W3, b3, W4, b4,
           Wo, bo, item_emb1, user_emb1, Wf, bf):
  x1 = x1.astype(jnp.int32)
  x2 = x2.astype(jnp.int32)
  xiu = _build_sc_k1()(x1, x2, item_emb, user_emb)
  gtab = _transcat(item_emb1.T, user_emb1.T)
  g1, g2 = _build_sc_k2()(x1, x2, gtab)
  bf16 = jnp.bfloat16
  w1ab = W1[:2 * D_IV].astype(bf16)
  w1c = W1[2 * D_IV:].astype(bf16)
  wf0 = Wf[0, 0]
  wo_s = (Wo[:, 0] * wf0)[None, :]
  wfg = Wf[1:, 0][None, :]
  c0 = (bo[0] * wf0 + bf[0])[None, None]
  out = _tc_mlp(xiu, x3.T, g1, g2, w1ab, w1c, b1[None, :],
                W2.astype(bf16), b2[None, :], W3.astype(bf16), b3[None, :],
                W4.astype(bf16), b4[None, :], wo_s, wfg, c0)
  return out.T


# speedup vs baseline: 1.7487x; 1.0971x over previous
import functools

import jax
import jax.numpy as jnp
from jax import lax
from jax.experimental import pallas as pl
from jax.experimental.pallas import tpu as pltpu
from jax.experimental.pallas import tpu_sc as plsc

B = 16384
N_TAB = 100000
D_IV = 128
D_GMF = 64
D_SIDE = 358
BM = 512
NC = 2
NS = 16
NW = NC * NS
BPW = B // NW
CH = 64
NCH = BPW // CH
NB = 2
HB = 8192
TPW = D_GMF // (NW // 2)


def _sc_pipeline(gather_fn, write_fn):
  pend_g = [None] * NB
  pend_w = [None] * NB

  def fire_writes(c):
    b = c % NB
    for g in pend_g[b]:
      g.wait()
    pend_w[b] = write_fn(c, b)

  for c in range(NCH):
    b = c % NB
    if pend_w[b] is not None:
      for w in pend_w[b]:
        w.wait()
    pend_g[b] = gather_fn(c, b)
    if c >= 1:
      fire_writes(c - 1)
  fire_writes(NCH - 1)
  for pw in pend_w:
    if pw is not None:
      for w in pw:
        w.wait()


@functools.lru_cache(maxsize=1)
def _build_sc_k1():
  mesh = plsc.VectorSubcoreMesh(core_axis_name="c", subcore_axis_name="s")

  @functools.partial(
      pl.kernel,
      out_type=jax.ShapeDtypeStruct((B, 2 * D_IV), jnp.float32),
      mesh=mesh,
      scratch_types=[
          pltpu.VMEM((BPW,), jnp.int32),
          pltpu.VMEM((BPW,), jnp.int32),
          pltpu.VMEM((NB, CH, D_IV), jnp.float32),
          pltpu.VMEM((NB, CH, D_IV), jnp.float32),
          pltpu.SemaphoreType.DMA,
          pltpu.SemaphoreType.DMA,
      ],
  )
  def k1(x1_hbm, x2_hbm, item_hbm, user_hbm, xiu_out,
         idx1_v, idx2_v, iv_v, uv_v, sg, sw):
    wid = lax.axis_index("s") * NC + lax.axis_index("c")
    base = wid * BPW
    pltpu.sync_copy(x1_hbm.at[pl.ds(base, BPW)], idx1_v)
    pltpu.sync_copy(x2_hbm.at[pl.ds(base, BPW)], idx2_v)

    def gather(c, b):
      loc = c * CH
      return (
          pltpu.async_copy(item_hbm.at[idx1_v.at[pl.ds(loc, CH)]],
                           iv_v.at[b], sg),
          pltpu.async_copy(user_hbm.at[idx2_v.at[pl.ds(loc, CH)]],
                           uv_v.at[b], sg),
      )

    def write(c, b):
      off = base + c * CH
      return (
          pltpu.async_copy(iv_v.at[b],
                           xiu_out.at[pl.ds(off, CH), pl.ds(0, D_IV)], sw),
          pltpu.async_copy(uv_v.at[b],
                           xiu_out.at[pl.ds(off, CH), pl.ds(D_IV, D_IV)],
                           sw),
      )

    _sc_pipeline(gather, write)

  return k1


@functools.lru_cache(maxsize=1)
def _build_sc_gmf():
  mesh = plsc.VectorSubcoreMesh(core_axis_name="c", subcore_axis_name="s")

  @functools.partial(
      pl.kernel,
      out_type=jax.ShapeDtypeStruct((2 * D_GMF, B), jnp.float32),
      mesh=mesh,
      scratch_types=[
          pltpu.VMEM((N_TAB,), jnp.float32),
          pltpu.VMEM((B,), jnp.int32),
          pltpu.VMEM((HB,), jnp.float32),
          pltpu.SemaphoreType.DMA,
      ],
      compiler_params=pltpu.CompilerParams(needs_layout_passes=False),
  )
  def gmf(xx_hbm, it_hbm, ut_hbm, o1, row_v, idx_v, out_v, sem):
    wid = lax.axis_index("s") * NC + lax.axis_index("c")
    is_item = wid < NW // 2
    lw = lax.rem(wid, NW // 2)
    ioff = jnp.where(is_item, 0, B).astype(jnp.int32)
    pltpu.sync_copy(xx_hbm.at[pl.ds(ioff, B)], idx_v)

    for t in range(TPW):
      d = lw * TPW + t

      @pl.when(is_item)
      def _():
        pltpu.sync_copy(it_hbm.at[d], row_v)

      @pl.when(jnp.logical_not(is_item))
      def _():
        pltpu.sync_copy(ut_hbm.at[d], row_v)

      dout = d + jnp.where(is_item, 0, D_GMF).astype(jnp.int32)
      for half in range(B // HB):
        base = half * HB

        def body(i, _):
          idx16 = idx_v[pl.ds(base + i * 16, 16)]
          out_v[pl.ds(i * 16, 16)] = plsc.load_gather(row_v, [idx16])
          return 0

        lax.fori_loop(0, HB // 16, body, 0)
        pltpu.sync_copy(out_v, o1.at[dout, pl.ds(base, HB)])

  return gmf


def _mlp_body(xiu_ref, x3t_ref, gv_ref,
              w1ab_ref, w1c_ref, b1_ref, w2_ref, b2_ref, w3_ref, b3_ref,
              w4_ref, b4_ref, wo_ref, wfg_ref, c0_ref, out_ref):
  f32 = jnp.float32
  bf16 = jnp.bfloat16
  h = jnp.dot(xiu_ref[...].astype(bf16), w1ab_ref[...],
              preferred_element_type=f32)
  h = h + lax.dot_general(x3t_ref[...].astype(bf16), w1c_ref[...],
                          (((0,), (0,)), ((), ())),
                          preferred_element_type=f32)
  h = jnp.maximum(h + b1_ref[...], 0.0).astype(bf16)
  h = jnp.maximum(jnp.dot(h, w2_ref[...], preferred_element_type=f32)
                  + b2_ref[...], 0.0).astype(bf16)
  h = jnp.maximum(jnp.dot(h, w3_ref[...], preferred_element_type=f32)
                  + b3_ref[...], 0.0).astype(bf16)
  h = jnp.maximum(jnp.dot(h, w4_ref[...], preferred_element_type=f32)
                  + b4_ref[...], 0.0)
  mlp = jnp.sum(h * wo_ref[...], axis=1, keepdims=True)
  gv = gv_ref[...]
  g = jnp.sum(gv[:D_GMF] * gv[D_GMF:] * wfg_ref[...], axis=0,
              keepdims=True)
  z = jnp.transpose(mlp + c0_ref[...]) + g
  out_ref[...] = 1.0 / (1.0 + jnp.exp(-z))


def _tc_mlp(xiu, x3t, gv, w1ab, w1c, b1r, w2, b2r, w3, b3r, w4, b4r,
            wo_s, wfg, c0, interpret=False):
  h1, h2, h3, h4 = 1024, 1024, 512, 256
  return pl.pallas_call(
      _mlp_body,
      grid=(B // BM,),
      in_specs=[
          pl.BlockSpec((BM, 2 * D_IV), lambda i: (i, 0)),
          pl.BlockSpec((D_SIDE, BM), lambda i: (0, i)),
          pl.BlockSpec((2 * D_GMF, BM), lambda i: (0, i)),
          pl.BlockSpec((2 * D_IV, h1), lambda i: (0, 0)),
          pl.BlockSpec((D_SIDE, h1), lambda i: (0, 0)),
          pl.BlockSpec((1, h1), lambda i: (0, 0)),
          pl.BlockSpec((h1, h2), lambda i: (0, 0)),
          pl.BlockSpec((1, h2), lambda i: (0, 0)),
          pl.BlockSpec((h2, h3), lambda i: (0, 0)),
          pl.BlockSpec((1, h3), lambda i: (0, 0)),
          pl.BlockSpec((h3, h4), lambda i: (0, 0)),
          pl.BlockSpec((1, h4), lambda i: (0, 0)),
          pl.BlockSpec((1, h4), lambda i: (0, 0)),
          pl.BlockSpec((D_GMF, 1), lambda i: (0, 0)),
          pl.BlockSpec((1, 1), lambda i: (0, 0)),
      ],
      out_specs=pl.BlockSpec((1, BM), lambda i: (0, i)),
      out_shape=jax.ShapeDtypeStruct((1, B), jnp.float32),
      compiler_params=pltpu.CompilerParams(
          dimension_semantics=("arbitrary",)),
      interpret=interpret,
  )(xiu, x3t, gv, w1ab, w1c, b1r, w2, b2r, w3, b3r, w4, b4r,
    wo_s, wfg, c0)


def kernel(x1, x2, x3, item_emb, user_emb, W1, b1, W2, b2, W3, b3, W4, b4,
           Wo, bo, item_emb1, user_emb1, Wf, bf):
  x1 = x1.astype(jnp.int32)
  x2 = x2.astype(jnp.int32)
  xiu = _build_sc_k1()(x1, x2, item_emb, user_emb)
  xx = jnp.concatenate([x1, x2])
  gv = _build_sc_gmf()(xx, item_emb1.T, user_emb1.T)
  bf16 = jnp.bfloat16
  w1ab = W1[:2 * D_IV].astype(bf16)
  w1c = W1[2 * D_IV:].astype(bf16)
  wf0 = Wf[0, 0]
  wo_s = (Wo[:, 0] * wf0)[None, :]
  wfg = Wf[1:, :1]
  c0 = (bo[0] * wf0 + bf[0])[None, None]
  out = _tc_mlp(xiu, x3.T, gv, w1ab, w1c, b1[None, :],
                W2.astype(bf16), b2[None, :], W3.astype(bf16), b3[None, :],
                W4.astype(bf16), b4[None, :], wo_s, wfg, c0)
  return out.T


# speedup vs baseline: 1.7647x; 1.0091x over previous
import functools

import jax
import jax.numpy as jnp
from jax import lax
from jax.experimental import pallas as pl
from jax.experimental.pallas import tpu as pltpu
from jax.experimental.pallas import tpu_sc as plsc

B = 16384
N_TAB = 100000
D_IV = 128
D_GMF = 64
D_SIDE = 358
BM = 1024
NC = 2
NS = 16
NW = NC * NS
BPW = B // NW
CH = 64
NCH = BPW // CH
NB = 2
HB = 4096
TPW = D_GMF // (NW // 2)


def _sc_pipeline(gather_fn, write_fn):
  pend_g = [None] * NB
  pend_w = [None] * NB

  def fire_writes(c):
    b = c % NB
    for g in pend_g[b]:
      g.wait()
    pend_w[b] = write_fn(c, b)

  for c in range(NCH):
    b = c % NB
    if pend_w[b] is not None:
      for w in pend_w[b]:
        w.wait()
    pend_g[b] = gather_fn(c, b)
    if c >= 1:
      fire_writes(c - 1)
  fire_writes(NCH - 1)
  for pw in pend_w:
    if pw is not None:
      for w in pw:
        w.wait()


@functools.lru_cache(maxsize=1)
def _build_sc_k1():
  mesh = plsc.VectorSubcoreMesh(core_axis_name="c", subcore_axis_name="s")

  @functools.partial(
      pl.kernel,
      out_type=jax.ShapeDtypeStruct((B, 2 * D_IV), jnp.float32),
      mesh=mesh,
      scratch_types=[
          pltpu.VMEM((BPW,), jnp.int32),
          pltpu.VMEM((BPW,), jnp.int32),
          pltpu.VMEM((NB, CH, D_IV), jnp.float32),
          pltpu.VMEM((NB, CH, D_IV), jnp.float32),
          pltpu.SemaphoreType.DMA,
          pltpu.SemaphoreType.DMA,
      ],
  )
  def k1(x1_hbm, x2_hbm, item_hbm, user_hbm, xiu_out,
         idx1_v, idx2_v, iv_v, uv_v, sg, sw):
    wid = lax.axis_index("s") * NC + lax.axis_index("c")
    base = wid * BPW
    pltpu.sync_copy(x1_hbm.at[pl.ds(base, BPW)], idx1_v)
    pltpu.sync_copy(x2_hbm.at[pl.ds(base, BPW)], idx2_v)

    def gather(c, b):
      loc = c * CH
      return (
          pltpu.async_copy(item_hbm.at[idx1_v.at[pl.ds(loc, CH)]],
                           iv_v.at[b], sg),
          pltpu.async_copy(user_hbm.at[idx2_v.at[pl.ds(loc, CH)]],
                           uv_v.at[b], sg),
      )

    def write(c, b):
      off = base + c * CH
      return (
          pltpu.async_copy(iv_v.at[b],
                           xiu_out.at[pl.ds(off, CH), pl.ds(0, D_IV)], sw),
          pltpu.async_copy(uv_v.at[b],
                           xiu_out.at[pl.ds(off, CH), pl.ds(D_IV, D_IV)],
                           sw),
      )

    _sc_pipeline(gather, write)

  return k1


@functools.lru_cache(maxsize=1)
def _build_sc_gmf():
  mesh = plsc.VectorSubcoreMesh(core_axis_name="c", subcore_axis_name="s")

  @functools.partial(
      pl.kernel,
      out_type=jax.ShapeDtypeStruct((2 * D_GMF, B), jnp.float32),
      mesh=mesh,
      scratch_types=[
          pltpu.VMEM((N_TAB,), jnp.float32),
          pltpu.VMEM((B,), jnp.int32),
          pltpu.VMEM((2, HB), jnp.float32),
          pltpu.SemaphoreType.DMA,
      ],
      compiler_params=pltpu.CompilerParams(needs_layout_passes=False),
  )
  def gmf(xx_hbm, it_hbm, ut_hbm, o1, row_v, idx_v, out_v, sem):
    wid = lax.axis_index("s") * NC + lax.axis_index("c")
    is_item = wid < NW // 2
    lw = lax.rem(wid, NW // 2)
    ioff = jnp.where(is_item, 0, B).astype(jnp.int32)
    pltpu.sync_copy(xx_hbm.at[pl.ds(ioff, B)], idx_v)
    pending = [None, None]

    for t in range(TPW):
      d = lw * TPW + t

      @pl.when(is_item)
      def _():
        pltpu.sync_copy(it_hbm.at[d], row_v)

      @pl.when(jnp.logical_not(is_item))
      def _():
        pltpu.sync_copy(ut_hbm.at[d], row_v)

      dout = d + jnp.where(is_item, 0, D_GMF).astype(jnp.int32)
      for half in range(B // HB):
        base = half * HB
        nb = (t * (B // HB) + half) % 2
        if pending[nb] is not None:
          pending[nb].wait()

        def body(i, _):
          idx16 = idx_v[pl.ds(base + i * 16, 16)]
          out_v[nb, pl.ds(i * 16, 16)] = plsc.load_gather(row_v, [idx16])
          return 0

        lax.fori_loop(0, HB // 16, body, 0)
        pending[nb] = pltpu.async_copy(
            out_v.at[nb], o1.at[dout, pl.ds(base, HB)], sem)
    for p in pending:
      if p is not None:
        p.wait()

  return gmf


def _mlp_body(xiu_ref, x3t_ref, gv_ref,
              w1ab_ref, w1c_ref, b1_ref, w2_ref, b2_ref, w3_ref, b3_ref,
              w4_ref, b4_ref, wo_ref, wfg_ref, c0_ref, out_ref):
  f32 = jnp.float32
  bf16 = jnp.bfloat16
  h = jnp.dot(xiu_ref[...].astype(bf16), w1ab_ref[...],
              preferred_element_type=f32)
  h = h + lax.dot_general(x3t_ref[...].astype(bf16), w1c_ref[...],
                          (((0,), (0,)), ((), ())),
                          preferred_element_type=f32)
  h = jnp.maximum(h + b1_ref[...], 0.0).astype(bf16)
  h = jnp.maximum(jnp.dot(h, w2_ref[...], preferred_element_type=f32)
                  + b2_ref[...], 0.0).astype(bf16)
  h = jnp.maximum(jnp.dot(h, w3_ref[...], preferred_element_type=f32)
                  + b3_ref[...], 0.0).astype(bf16)
  h = jnp.maximum(jnp.dot(h, w4_ref[...], preferred_element_type=f32)
                  + b4_ref[...], 0.0)
  mlp = jnp.sum(h * wo_ref[...], axis=1, keepdims=True)
  gv = gv_ref[...]
  g = jnp.sum(gv[:D_GMF] * gv[D_GMF:] * wfg_ref[...], axis=0,
              keepdims=True)
  z = jnp.transpose(mlp + c0_ref[...]) + g
  out_ref[...] = 1.0 / (1.0 + jnp.exp(-z))


def _tc_mlp(xiu, x3t, gv, w1ab, w1c, b1r, w2, b2r, w3, b3r, w4, b4r,
            wo_s, wfg, c0, interpret=False):
  h1, h2, h3, h4 = 1024, 1024, 512, 256
  return pl.pallas_call(
      _mlp_body,
      grid=(B // BM,),
      in_specs=[
          pl.BlockSpec((BM, 2 * D_IV), lambda i: (i, 0)),
          pl.BlockSpec((D_SIDE, BM), lambda i: (0, i)),
          pl.BlockSpec((2 * D_GMF, BM), lambda i: (0, i)),
          pl.BlockSpec((2 * D_IV, h1), lambda i: (0, 0)),
          pl.BlockSpec((D_SIDE, h1), lambda i: (0, 0)),
          pl.BlockSpec((1, h1), lambda i: (0, 0)),
          pl.BlockSpec((h1, h2), lambda i: (0, 0)),
          pl.BlockSpec((1, h2), lambda i: (0, 0)),
          pl.BlockSpec((h2, h3), lambda i: (0, 0)),
          pl.BlockSpec((1, h3), lambda i: (0, 0)),
          pl.BlockSpec((h3, h4), lambda i: (0, 0)),
          pl.BlockSpec((1, h4), lambda i: (0, 0)),
          pl.BlockSpec((1, h4), lambda i: (0, 0)),
          pl.BlockSpec((D_GMF, 1), lambda i: (0, 0)),
          pl.BlockSpec((1, 1), lambda i: (0, 0)),
      ],
      out_specs=pl.BlockSpec((1, BM), lambda i: (0, i)),
      out_shape=jax.ShapeDtypeStruct((1, B), jnp.float32),
      compiler_params=pltpu.CompilerParams(
          dimension_semantics=("arbitrary",)),
      interpret=interpret,
  )(xiu, x3t, gv, w1ab, w1c, b1r, w2, b2r, w3, b3r, w4, b4r,
    wo_s, wfg, c0)


def kernel(x1, x2, x3, item_emb, user_emb, W1, b1, W2, b2, W3, b3, W4, b4,
           Wo, bo, item_emb1, user_emb1, Wf, bf):
  x1 = x1.astype(jnp.int32)
  x2 = x2.astype(jnp.int32)
  xiu = _build_sc_k1()(x1, x2, item_emb, user_emb)
  xx = jnp.concatenate([x1, x2])
  gv = _build_sc_gmf()(xx, item_emb1.T, user_emb1.T)
  bf16 = jnp.bfloat16
  w1ab = W1[:2 * D_IV].astype(bf16)
  w1c = W1[2 * D_IV:].astype(bf16)
  wf0 = Wf[0, 0]
  wo_s = (Wo[:, 0] * wf0)[None, :]
  wfg = Wf[1:, :1]
  c0 = (bo[0] * wf0 + bf[0])[None, None]
  out = _tc_mlp(xiu, x3.T, gv, w1ab, w1c, b1[None, :],
                W2.astype(bf16), b2[None, :], W3.astype(bf16), b3[None, :],
                W4.astype(bf16), b4[None, :], wo_s, wfg, c0)
  return out.T
